# baseline (speedup 1.0000x reference)
import jax
import jax.numpy as jnp
from jax import lax
from jax.experimental import pallas as pl
from jax.experimental.pallas import tpu as pltpu
from jax.experimental.pallas import tpu_sc as plsc

VOCAB = 1000000
HID = 64
MAXPOS = 512
TYPEV = 2
B = 1024
S = 200

NUM_CORES = 2
NUM_SUBCORES = 16
NW = NUM_CORES * NUM_SUBCORES
N_TOK = B * S
PER_W = N_TOK // NW
CHUNK = 400
N_CHUNK = PER_W // CHUNK


SPLIT = 512000


def _word_gather_body(tok_hbm, lo_hbm, hi_hbm, out_hbm, idx_all, rows_a,
                      rows_b, sem_a, sem_b):
    wid = lax.axis_index("s") * NUM_CORES + lax.axis_index("c")
    base0 = wid * PER_W
    pltpu.sync_copy(tok_hbm.at[pl.ds(base0, PER_W)], idx_all)
    bufs = (rows_a, rows_b)
    sems = (sem_a, sem_b)

    def fire_chunk(c, buf, sem):
        def fire(g, carry):
            vec = idx_all[pl.ds(c * CHUNK + g * 16, 16)]
            for j in range(16):
                r = vec[j]
                dst = buf.at[pl.ds(g * 16 + j, 1)]

                @pl.when(r < SPLIT)
                def _():
                    pltpu.async_copy(lo_hbm.at[pl.ds(r, 1)], dst, sem)

                @pl.when(r >= SPLIT)
                def _():
                    pltpu.async_copy(hi_hbm.at[pl.ds(r - SPLIT, 1)], dst, sem)

            return carry

        lax.fori_loop(0, CHUNK // 16, fire, None)

    fire_chunk(0, bufs[0], sems[0])
    for c in range(N_CHUNK):
        if c + 1 < N_CHUNK:
            fire_chunk(c + 1, bufs[(c + 1) % 2], sems[(c + 1) % 2])
        pltpu.make_async_copy(
            lo_hbm.at[pl.ds(0, CHUNK)], bufs[c % 2], sems[c % 2]
        ).wait()
        pltpu.sync_copy(bufs[c % 2], out_hbm.at[pl.ds(base0 + c * CHUNK, CHUNK)])


@jax.jit
def _word_gather(tok_flat, table_lo, table_hi):
    mesh = plsc.VectorSubcoreMesh(core_axis_name="c", subcore_axis_name="s")
    return pl.kernel(
        _word_gather_body,
        out_type=jax.ShapeDtypeStruct((N_TOK, HID), jnp.float32),
        mesh=mesh,
        scratch_types=[
            pltpu.VMEM((PER_W,), jnp.int32),
            pltpu.VMEM((CHUNK, HID), jnp.float32),
            pltpu.VMEM((CHUNK, HID), jnp.float32),
            pltpu.SemaphoreType.DMA,
            pltpu.SemaphoreType.DMA,
        ],
        compiler_params=pltpu.CompilerParams(use_tc_tiling_on_sc=True),
    )(tok_flat, table_lo, table_hi)


LB = 10240


def _repack_body(wt_ref, out_ref):
    out_ref[...] = wt_ref[...].T


@jax.jit
def _repack(w_t):
    return pl.pallas_call(
        _repack_body,
        grid=(SPLIT // LB,),
        in_specs=[pl.BlockSpec((HID, LB), lambda i: (0, i))],
        out_specs=pl.BlockSpec((LB, HID), lambda i: (i, 0)),
        out_shape=jax.ShapeDtypeStruct((SPLIT, HID), jnp.float32),
    )(w_t)


SS = 20


def _bcast_body(pos_ref, type_ref, pos_out, type_out):
    tbc = jnp.broadcast_to(type_ref[0], (HID, B))
    for i in range(SS):
        pos_out[i] = jnp.broadcast_to(pos_ref[i], (HID, B))
        type_out[i] = tbc


@jax.jit
def _broadcasts(w_pos_col, w_type_col):
    out_shape = jax.ShapeDtypeStruct((S, HID, B), jnp.float32)
    return pl.pallas_call(
        _bcast_body,
        grid=(S // SS,),
        in_specs=[
            pl.BlockSpec((SS, HID, 1), lambda i: (i, 0, 0)),
            pl.BlockSpec((1, HID, 1), lambda i: (0, 0, 0)),
        ],
        out_specs=[
            pl.BlockSpec((SS, HID, B), lambda i: (i, 0, 0)),
            pl.BlockSpec((SS, HID, B), lambda i: (i, 0, 0)),
        ],
        out_shape=[out_shape, out_shape],
    )(w_pos_col, w_type_col)


def kernel(token_ids, W_word, W_pos, W_type):
    tok_sb = token_ids.T.reshape(N_TOK).astype(jnp.int32)
    table_lo = _repack(W_word.T)
    pos_t, type_t = _broadcasts(
        W_pos[:S].reshape(S, HID, 1), W_type[0].reshape(1, HID, 1)
    )
    out_rows = _word_gather(tok_sb, table_lo, W_word[SPLIT:])
    word = out_rows.reshape(S, B, HID).transpose(1, 0, 2)
    return (word, pos_t.transpose(2, 0, 1), type_t.transpose(2, 0, 1))


# speedup vs baseline: 1.3188x; 1.3188x over previous
import jax
import jax.numpy as jnp
from jax import lax
from jax.experimental import pallas as pl
from jax.experimental.pallas import tpu as pltpu
from jax.experimental.pallas import tpu_sc as plsc

VOCAB = 1000000
HID = 64
MAXPOS = 512
TYPEV = 2
B = 1024
S = 200

NUM_CORES = 2
NUM_SUBCORES = 16
NW = NUM_CORES * NUM_SUBCORES
N_TOK = B * S
PER_W = N_TOK // NW
CHUNK = 400
N_CHUNK = PER_W // CHUNK


def _word_gather_body(tok_hbm, table_hbm, out_hbm, idx_all, rows_a, rows_b,
                      sem_a, sem_b):
    wid = lax.axis_index("s") * NUM_CORES + lax.axis_index("c")
    base0 = wid * PER_W
    pltpu.sync_copy(tok_hbm.at[pl.ds(base0, PER_W)], idx_all)
    bufs = (rows_a, rows_b)
    sems = (sem_a, sem_b)

    def fire_chunk(c, buf, sem):
        def fire(g, carry):
            vec = idx_all[pl.ds(c * CHUNK + g * 16, 16)]
            for j in range(16):
                pltpu.async_copy(
                    table_hbm.at[pl.ds(vec[j], 1)],
                    buf.at[pl.ds(g * 16 + j, 1)],
                    sem,
                )
            return carry

        lax.fori_loop(0, CHUNK // 16, fire, None)

    fire_chunk(0, bufs[0], sems[0])
    for c in range(N_CHUNK):
        if c + 1 < N_CHUNK:
            fire_chunk(c + 1, bufs[(c + 1) % 2], sems[(c + 1) % 2])
        pltpu.make_async_copy(
            table_hbm.at[pl.ds(0, CHUNK)], bufs[c % 2], sems[c % 2]
        ).wait()
        pltpu.sync_copy(bufs[c % 2], out_hbm.at[pl.ds(base0 + c * CHUNK, CHUNK)])


@jax.jit
def _word_gather(tok_flat, table_lin):
    mesh = plsc.VectorSubcoreMesh(core_axis_name="c", subcore_axis_name="s")
    return pl.kernel(
        _word_gather_body,
        out_type=jax.ShapeDtypeStruct((N_TOK, HID), jnp.float32),
        mesh=mesh,
        scratch_types=[
            pltpu.VMEM((PER_W,), jnp.int32),
            pltpu.VMEM((CHUNK, HID), jnp.float32),
            pltpu.VMEM((CHUNK, HID), jnp.float32),
            pltpu.SemaphoreType.DMA,
            pltpu.SemaphoreType.DMA,
        ],
        compiler_params=pltpu.CompilerParams(use_tc_tiling_on_sc=True),
    )(tok_flat, table_lin)


LB = 16384


def _repack_body(wt_ref, out_ref):
    out_ref[...] = wt_ref[...].T


@jax.jit
def _repack(w_t):
    return pl.pallas_call(
        _repack_body,
        grid=(pl.cdiv(VOCAB, LB),),
        in_specs=[pl.BlockSpec((HID, LB), lambda i: (0, i))],
        out_specs=pl.BlockSpec((LB, HID), lambda i: (i, 0)),
        out_shape=jax.ShapeDtypeStruct((VOCAB, HID), jnp.float32),
    )(w_t)


SS = 20


def _bcast_body(pos_ref, type_ref, pos_out, type_out):
    tbc = jnp.broadcast_to(type_ref[0], (HID, B))
    for i in range(SS):
        pos_out[i] = jnp.broadcast_to(pos_ref[i], (HID, B))
        type_out[i] = tbc


@jax.jit
def _broadcasts(w_pos_col, w_type_col):
    out_shape = jax.ShapeDtypeStruct((S, HID, B), jnp.float32)
    return pl.pallas_call(
        _bcast_body,
        grid=(S // SS,),
        in_specs=[
            pl.BlockSpec((SS, HID, 1), lambda i: (i, 0, 0)),
            pl.BlockSpec((1, HID, 1), lambda i: (0, 0, 0)),
        ],
        out_specs=[
            pl.BlockSpec((SS, HID, B), lambda i: (i, 0, 0)),
            pl.BlockSpec((SS, HID, B), lambda i: (i, 0, 0)),
        ],
        out_shape=[out_shape, out_shape],
    )(w_pos_col, w_type_col)


def kernel(token_ids, W_word, W_pos, W_type):
    tok_sb = token_ids.T.reshape(N_TOK).astype(jnp.int32)
    table_lin = _repack(W_word.T)
    pos_t, type_t = _broadcasts(
        W_pos[:S].reshape(S, HID, 1), W_type[0].reshape(1, HID, 1)
    )
    out_rows = _word_gather(tok_sb, table_lin)
    word = out_rows.reshape(S, B, HID).transpose(1, 0, 2)
    return (word, pos_t.transpose(2, 0, 1), type_t.transpose(2, 0, 1))
